# Initial kernel scaffold; baseline (speedup 1.0000x reference)
import jax
import jax.numpy as jnp
from jax.experimental import pallas as pl


def kernel(x, gate_w, gate_b, w1, w2, w3):
    raise NotImplementedError("write your pallas kernel here")


# speedup vs baseline: 2.3580x; 2.3580x over previous
import functools

import jax
import jax.numpy as jnp
from jax.experimental import pallas as pl
from jax.experimental.pallas import tpu as pltpu

T = 512
H = 1024
I = 512
E = 16
TOP_K = 2


def _moe_body(x_ref, gw_ref, gb_ref, w1_ref, w2_ref, w3_ref, out_ref, comb_ref):
    e = pl.program_id(0)

    @pl.when(e == 0)
    def _router():
        x = x_ref[...]
        logits = jax.lax.dot_general(
            x, gw_ref[...], (((1,), (0,)), ((), ())),
            preferred_element_type=jnp.float32) + gb_ref[...]
        z = logits - jnp.max(logits, axis=1, keepdims=True)
        ez = jnp.exp(z)
        rw = ez / jnp.sum(ez, axis=1, keepdims=True)
        m1 = jnp.max(rw, axis=1, keepdims=True)
        rw_wo1 = jnp.where(rw == m1, -jnp.inf, rw)
        m2 = jnp.max(rw_wo1, axis=1, keepdims=True)
        mask = rw >= m2
        picked = jnp.where(mask, rw, 0.0)
        comb_ref[...] = picked / jnp.sum(picked, axis=1, keepdims=True)
        out_ref[...] = jnp.zeros_like(out_ref)

    x = x_ref[...]
    w1e = w1_ref[0]
    w3e = w3_ref[0]
    h1 = jax.lax.dot_general(x, w1e, (((1,), (1,)), ((), ())),
                             preferred_element_type=jnp.float32)
    h3 = jax.lax.dot_general(x, w3e, (((1,), (1,)), ((), ())),
                             preferred_element_type=jnp.float32)
    h = h1 * jax.nn.sigmoid(h1) * h3
    y = jax.lax.dot_general(h, w2_ref[0], (((1,), (1,)), ((), ())),
                            preferred_element_type=jnp.float32)
    lane = jax.lax.broadcasted_iota(jnp.int32, (T, E), 1)
    ce = jnp.sum(jnp.where(lane == e, comb_ref[...], 0.0), axis=1, keepdims=True)
    out_ref[...] += ce * y


@functools.partial(jax.jit, static_argnames=("interpret",))
def kernel(x, gate_w, gate_b, w1, w2, w3, interpret=False):
    gb2 = gate_b.reshape(1, E)
    return pl.pallas_call(
        _moe_body,
        grid=(E,),
        in_specs=[
            pl.BlockSpec((T, H), lambda e: (0, 0)),
            pl.BlockSpec((H, E), lambda e: (0, 0)),
            pl.BlockSpec((1, E), lambda e: (0, 0)),
            pl.BlockSpec((1, I, H), lambda e: (e, 0, 0)),
            pl.BlockSpec((1, H, I), lambda e: (e, 0, 0)),
            pl.BlockSpec((1, I, H), lambda e: (e, 0, 0)),
        ],
        out_specs=pl.BlockSpec((T, H), lambda e: (0, 0)),
        out_shape=jax.ShapeDtypeStruct((T, H), jnp.float32),
        scratch_shapes=[pltpu.VMEM((T, E), jnp.float32)],
        interpret=interpret,
    )(x, gate_w, gb2, w1, w2, w3)
